# Initial kernel scaffold; baseline (speedup 1.0000x reference)
import functools

import jax
import jax.numpy as jnp
from jax import lax
from jax.experimental import pallas as pl
from jax.experimental.pallas import tpu as pltpu
from jax.experimental.pallas import tpu_sc as plsc

N_NODES = 10000
N_EDGES = 320000
D_NODE = 128
DE = 16
G = 32
T_DIM = 128
EPS = 1e-5

NC = 2
NS = 16
NW = NC * NS
L = 16

E_ROWS = 2560
E_PAD = E_ROWS * 128
ROWS_PER_W = E_ROWS // NW
SUB_ROWS = 16
N_SUB = ROWS_PER_W // SUB_ROWS
SUB_E = SUB_ROWS * 128

N_ACC = 10240
SC_ROWS_PER_T = E_ROWS // NS
SC_SUB_ROWS = 16
SC_N_SUB = SC_ROWS_PER_T // SC_SUB_ROWS

BC = 2048


def _node_tf_body(h_ref, ws_ref, bs_ref, wd_ref, bd_ref, ns_ref, nd_ref):
    h = h_ref[...]
    inv = 1.0 / jnp.sqrt(float(D_NODE))
    ns_ref[...] = jnp.dot(h, ws_ref[...], preferred_element_type=jnp.float32) * inv + bs_ref[...]
    nd_ref[...] = jnp.dot(h, wd_ref[...], preferred_element_type=jnp.float32) * inv + bd_ref[...]


def _tmlp_body(t_ref, w1_ref, b1_ref, w2_ref, b2_ref, ss_ref):
    half = T_DIM // 2
    k = lax.broadcasted_iota(jnp.float32, (G, half), 1)
    freqs = jnp.exp(k * (-jnp.log(10000.0) / half))
    args = t_ref[...] * freqs
    temb = jnp.concatenate([jnp.cos(args), jnp.sin(args)], axis=1)
    x = jnp.dot(temb, w1_ref[...], preferred_element_type=jnp.float32) + b1_ref[...]
    x = x * (1.0 / (1.0 + jnp.exp(-x)))
    ss = jnp.dot(x, w2_ref[...], preferred_element_type=jnp.float32) + b2_ref[...]
    one0 = jnp.concatenate([jnp.ones((1, DE), jnp.float32), jnp.zeros((1, DE), jnp.float32)], axis=1)
    ss_ref[...] = ss + one0


def _gather_body(ns_hbm, nd_hbm, si_hbm, di_hbm, batch_hbm,
                 gs_hbm, gd_hbm, be_hbm,
                 idx_s, idx_d, rows_s, rows_d, be_v, batch_v, sem):
    wid = lax.axis_index("s") * NC + lax.axis_index("c")
    pltpu.sync_copy(batch_hbm, batch_v)

    def sub_step(sub, _):
        base_row = wid * ROWS_PER_W + sub * SUB_ROWS
        pltpu.sync_copy(si_hbm.at[pl.ds(base_row, SUB_ROWS)], idx_s)
        pltpu.sync_copy(di_hbm.at[pl.ds(base_row, SUB_ROWS)], idx_d)
        descs = []
        for j in range(SUB_ROWS):
            descs.append(pltpu.async_copy(
                ns_hbm.at[idx_s.at[j]], rows_s.at[pl.ds(j * 128, 128)], sem))
            descs.append(pltpu.async_copy(
                nd_hbm.at[idx_d.at[j]], rows_d.at[pl.ds(j * 128, 128)], sem))
        for j in range(SUB_ROWS):
            for g in range(128 // L):
                iv = idx_s[j, pl.ds(g * L, L)]
                be_v[j, pl.ds(g * L, L)] = plsc.load_gather(batch_v, [iv])
        for dsc in descs:
            dsc.wait()
        base_e = base_row * 128
        pltpu.sync_copy(rows_s, gs_hbm.at[pl.ds(base_e, SUB_E)])
        pltpu.sync_copy(rows_d, gd_hbm.at[pl.ds(base_e, SUB_E)])
        pltpu.sync_copy(be_v, be_hbm.at[pl.ds(base_row, SUB_ROWS)])
        return ()

    lax.fori_loop(0, N_SUB, sub_step, (), unroll=False)


def _edge_body(gs_ref, gd_ref, he_ref, be_ref, rv_ref, dist_ref, ss_ref,
               w1cat_ref, w2cat_ref, r_ref, s_ref,
               wnt_ref, bnt_ref, bt1_ref, bt2_ref, wet_ref, bet_ref,
               wsp1_ref, bsp1_ref, wsp2_ref, bsp2_ref, out_ref):
    f32 = jnp.float32
    dot = functools.partial(jnp.dot, preferred_element_type=f32)
    s = gs_ref[...]
    d = gd_ref[...]
    R = r_ref[...]
    S = s_ref[...]
    f1 = dot(dot(s, R) * dot(d, w1cat_ref[...]), S) * (1.0 / 16.0) + bt1_ref[...]
    fnt = dot(f1, wnt_ref[...]) * 0.25 + bnt_ref[...]
    et = dot(he_ref[...], wet_ref[...]) * 0.25 + bet_ref[...]
    f2 = dot(dot(fnt, R) * dot(et, w2cat_ref[...]), S) * (1.0 / 16.0) + bt2_ref[...]
    mu = jnp.mean(f2, axis=1, keepdims=True)
    c = f2 - mu
    var = jnp.mean(c * c, axis=1, keepdims=True)
    normed = c * lax.rsqrt(var + EPS)
    onehot = (be_ref[...] == lax.broadcasted_iota(jnp.int32, (BC, G), 1)).astype(f32)
    ssr = dot(onehot, ss_ref[...])
    normed = normed * ssr[:, :DE] + ssr[:, DE:]
    h = dot(normed, wsp1_ref[...]) + bsp1_ref[...]
    h = h * (1.0 / (1.0 + jnp.exp(-h)))
    sw = dot(h, wsp2_ref[...]) + bsp2_ref[...]
    dist = dist_ref[...]
    out_ref[...] = (sw / (dist * (dist + 1.0))) * rv_ref[...]


def _scatter_body(force_hbm, si_hbm, zeros_hbm, out_hbm,
                  acc, force_v, idx_v, sem):
    cid = lax.axis_index("c")
    tid = lax.axis_index("s")

    @pl.when(cid == 0)
    def _():
        pltpu.sync_copy(zeros_hbm.at[pl.ds(tid * 320, 320)],
                        acc.at[pl.ds(tid * 320, 320)])
        plsc.subcore_barrier()

        def sub_step(sub, _):
            base_row = tid * SC_ROWS_PER_T + sub * SC_SUB_ROWS
            pltpu.sync_copy(si_hbm.at[pl.ds(base_row, SC_SUB_ROWS)], idx_v)
            pltpu.sync_copy(force_hbm.at[pl.ds(base_row * 128, SC_SUB_ROWS * 128)],
                            force_v)
            for j in range(SC_SUB_ROWS):
                pltpu.sync_copy(force_v.at[pl.ds(j * 128, 128)],
                                acc.at[idx_v.at[j]], add=True)
            return ()

        lax.fori_loop(0, SC_N_SUB, sub_step, (), unroll=False)
        plsc.subcore_barrier()
        pltpu.sync_copy(acc.at[pl.ds(tid * 640, 640)],
                        out_hbm.at[pl.ds(tid * 640, 640)])


@jax.jit
def _run(h_node, h_edge, edge_index, relative_vec, distance, t, batch,
         W_src, b_src, W_dst, b_dst, w_tp1, b_tp1, W_nt, b_nt, W_et, b_et,
         w_tp2, b_tp2, W_t1, b_t1, W_t2, b_t2, W_sp1, b_sp1, W_sp2, b_sp2):
    f32 = jnp.float32

    node_src, node_dst = pl.pallas_call(
        _node_tf_body,
        grid=(10,),
        in_specs=[
            pl.BlockSpec((1000, D_NODE), lambda i: (i, 0)),
            pl.BlockSpec((D_NODE, DE), lambda i: (0, 0)),
            pl.BlockSpec((1, DE), lambda i: (0, 0)),
            pl.BlockSpec((D_NODE, DE), lambda i: (0, 0)),
            pl.BlockSpec((1, DE), lambda i: (0, 0)),
        ],
        out_specs=[
            pl.BlockSpec((1000, DE), lambda i: (i, 0)),
            pl.BlockSpec((1000, DE), lambda i: (i, 0)),
        ],
        out_shape=[
            jax.ShapeDtypeStruct((N_NODES, DE), f32),
            jax.ShapeDtypeStruct((N_NODES, DE), f32),
        ],
    )(h_node, W_src, b_src.reshape(1, DE), W_dst, b_dst.reshape(1, DE))

    ss_tab = pl.pallas_call(
        _tmlp_body,
        out_shape=jax.ShapeDtypeStruct((G, 2 * DE), f32),
    )(t.reshape(G, 1), W_t1, b_t1.reshape(1, T_DIM), W_t2, b_t2.reshape(1, 2 * DE))

    pad_e = E_PAD - N_EDGES
    src = jnp.concatenate([edge_index[0], jnp.zeros((pad_e,), jnp.int32)])
    dst = jnp.concatenate([edge_index[1], jnp.zeros((pad_e,), jnp.int32)])
    src3 = src.reshape(E_ROWS, 128)
    dst3 = dst.reshape(E_ROWS, 128)
    he_pad = jnp.concatenate([h_edge, jnp.zeros((pad_e, DE), f32)])
    rv_pad = jnp.concatenate([
        jnp.concatenate([relative_vec, jnp.zeros((N_EDGES, 1), f32)], axis=1),
        jnp.zeros((pad_e, 4), f32)])
    dist_pad = jnp.concatenate([distance, jnp.ones((pad_e, 1), f32)])

    mesh = plsc.VectorSubcoreMesh(core_axis_name="c", subcore_axis_name="s")
    gs, gd, be = pl.kernel(
        _gather_body,
        out_type=[
            jax.ShapeDtypeStruct((E_PAD, DE), f32),
            jax.ShapeDtypeStruct((E_PAD, DE), f32),
            jax.ShapeDtypeStruct((E_ROWS, 128), jnp.int32),
        ],
        mesh=mesh,
        scratch_types=[
            pltpu.VMEM((SUB_ROWS, 128), jnp.int32),
            pltpu.VMEM((SUB_ROWS, 128), jnp.int32),
            pltpu.VMEM((SUB_E, DE), f32),
            pltpu.VMEM((SUB_E, DE), f32),
            pltpu.VMEM((SUB_ROWS, 128), jnp.int32),
            pltpu.VMEM((N_NODES,), jnp.int32),
            pltpu.SemaphoreType.DMA,
        ],
    )(node_src, node_dst, src3, dst3, batch)

    w1cat = w_tp1.transpose(1, 0, 2).reshape(DE, DE * DE)
    w2cat = w_tp2.transpose(1, 0, 2).reshape(DE, DE * DE)
    R = jnp.kron(jnp.eye(DE, dtype=f32), jnp.ones((1, DE), f32))
    S = jnp.kron(jnp.ones((DE, 1), f32), jnp.eye(DE, dtype=f32))
    n_blk = E_PAD // BC
    force = pl.pallas_call(
        _edge_body,
        grid=(n_blk,),
        in_specs=[
            pl.BlockSpec((BC, DE), lambda i: (i, 0)),
            pl.BlockSpec((BC, DE), lambda i: (i, 0)),
            pl.BlockSpec((BC, DE), lambda i: (i, 0)),
            pl.BlockSpec((BC, 1), lambda i: (i, 0)),
            pl.BlockSpec((BC, 4), lambda i: (i, 0)),
            pl.BlockSpec((BC, 1), lambda i: (i, 0)),
            pl.BlockSpec((G, 2 * DE), lambda i: (0, 0)),
            pl.BlockSpec((DE, DE * DE), lambda i: (0, 0)),
            pl.BlockSpec((DE, DE * DE), lambda i: (0, 0)),
            pl.BlockSpec((DE, DE * DE), lambda i: (0, 0)),
            pl.BlockSpec((DE * DE, DE), lambda i: (0, 0)),
            pl.BlockSpec((DE, DE), lambda i: (0, 0)),
            pl.BlockSpec((1, DE), lambda i: (0, 0)),
            pl.BlockSpec((1, DE), lambda i: (0, 0)),
            pl.BlockSpec((1, DE), lambda i: (0, 0)),
            pl.BlockSpec((DE, DE), lambda i: (0, 0)),
            pl.BlockSpec((1, DE), lambda i: (0, 0)),
            pl.BlockSpec((DE, 32), lambda i: (0, 0)),
            pl.BlockSpec((1, 32), lambda i: (0, 0)),
            pl.BlockSpec((32, 1), lambda i: (0, 0)),
            pl.BlockSpec((1, 1), lambda i: (0, 0)),
        ],
        out_specs=pl.BlockSpec((BC, 4), lambda i: (i, 0)),
        out_shape=jax.ShapeDtypeStruct((E_PAD, 4), f32),
    )(gs, gd, he_pad, be.reshape(E_PAD, 1), rv_pad, dist_pad, ss_tab,
      w1cat, w2cat, R, S,
      W_nt, b_nt.reshape(1, DE), b_tp1.reshape(1, DE), b_tp2.reshape(1, DE),
      W_et, b_et.reshape(1, DE),
      W_sp1, b_sp1.reshape(1, 32), W_sp2, b_sp2.reshape(1, 1))

    dp = pl.kernel(
        _scatter_body,
        out_type=jax.ShapeDtypeStruct((N_ACC, 4), f32),
        mesh=mesh,
        scratch_types=[
            pltpu.VMEM_SHARED((N_ACC, 4), f32),
            pltpu.VMEM((SC_SUB_ROWS * 128, 4), f32),
            pltpu.VMEM((SC_SUB_ROWS, 128), jnp.int32),
            pltpu.SemaphoreType.DMA,
        ],
    )(force, src3, jnp.zeros((N_ACC, 4), f32))

    return dp[:N_NODES, :3]


def kernel(h_node, h_edge, pos, edge_index, relative_vec, distance, t, batch,
           W_src, b_src, W_dst, b_dst, w_tp1, b_tp1, W_nt, b_nt, W_et, b_et,
           w_tp2, b_tp2, W_t1, b_t1, W_t2, b_t2, W_sp1, b_sp1, W_sp2, b_sp2):
    return _run(h_node, h_edge, edge_index, relative_vec, distance, t, batch,
                W_src, b_src, W_dst, b_dst, w_tp1, b_tp1, W_nt, b_nt, W_et,
                b_et, w_tp2, b_tp2, W_t1, b_t1, W_t2, b_t2, W_sp1, b_sp1,
                W_sp2, b_sp2)


# speedup vs baseline: 4.4451x; 4.4451x over previous
import functools

import jax
import jax.numpy as jnp
from jax import lax
from jax.experimental import pallas as pl
from jax.experimental.pallas import tpu as pltpu
from jax.experimental.pallas import tpu_sc as plsc

N_NODES = 10000
N_EDGES = 320000
D_NODE = 128
DE = 16
DC = 48
G = 32
T_DIM = 128
EPS = 1e-5

NC = 2
NS = 16
NW = NC * NS

E_ROWS = 2560
E_PAD = E_ROWS * 128
ROWS_PER_W = E_ROWS // NW
SUB_ROWS = 8
N_SUB = ROWS_PER_W // SUB_ROWS
SUB_E = SUB_ROWS * 128

N_ACC = 10240
SC_ROWS_PER_T = E_ROWS // NS
SC_SUB_ROWS = 16
SC_N_SUB = SC_ROWS_PER_T // SC_SUB_ROWS

BC = 2048
FW = 8


def _node_tf_body(h_ref, batch_ref, ws_ref, wd_ref, bs_ref, bd_ref, ss_ref,
                  ncat_ref, nd_ref):
    h = h_ref[...]
    inv = 1.0 / jnp.sqrt(float(D_NODE))
    src = jnp.dot(h, ws_ref[...], preferred_element_type=jnp.float32) * inv + bs_ref[...]
    nd_ref[...] = jnp.dot(h, wd_ref[...], preferred_element_type=jnp.float32) * inv + bd_ref[...]
    onehot = (batch_ref[...] == lax.broadcasted_iota(jnp.int32, (h.shape[0], G), 1)).astype(jnp.float32)
    ss = jnp.dot(onehot, ss_ref[...], preferred_element_type=jnp.float32)
    ncat_ref[...] = jnp.concatenate([src, ss], axis=1)


def _tmlp_body(t_ref, w1_ref, b1_ref, w2_ref, b2_ref, ss_ref):
    half = T_DIM // 2
    k = lax.broadcasted_iota(jnp.int32, (G, half), 1).astype(jnp.float32)
    freqs = jnp.exp(k * (-jnp.log(10000.0) / half))
    args = t_ref[...] * freqs
    temb = jnp.concatenate([jnp.cos(args), jnp.sin(args)], axis=1)
    x = jnp.dot(temb, w1_ref[...], preferred_element_type=jnp.float32) + b1_ref[...]
    x = x * (1.0 / (1.0 + jnp.exp(-x)))
    ss = jnp.dot(x, w2_ref[...], preferred_element_type=jnp.float32) + b2_ref[...]
    one0 = jnp.concatenate([jnp.ones((1, DE), jnp.float32), jnp.zeros((1, DE), jnp.float32)], axis=1)
    ss_ref[...] = ss + one0


def _gather_body(nc_hbm, nd_hbm, si_hbm, di_hbm,
                 gs_hbm, gd_hbm,
                 idx_s, idx_d, rows_s, rows_d, sem):
    wid = lax.axis_index("s") * NC + lax.axis_index("c")

    def sub_step(sub, _):
        base_row = wid * ROWS_PER_W + sub * SUB_ROWS
        pltpu.sync_copy(si_hbm.at[pl.ds(base_row, SUB_ROWS)], idx_s)
        pltpu.sync_copy(di_hbm.at[pl.ds(base_row, SUB_ROWS)], idx_d)
        descs = []
        for j in range(SUB_ROWS):
            descs.append(pltpu.async_copy(
                nc_hbm.at[idx_s.at[j]], rows_s.at[pl.ds(j * 128, 128)], sem))
            descs.append(pltpu.async_copy(
                nd_hbm.at[idx_d.at[j]], rows_d.at[pl.ds(j * 128, 128)], sem))
        for dsc in descs:
            dsc.wait()
        base_e = base_row * 128
        pltpu.sync_copy(rows_s, gs_hbm.at[pl.ds(base_e, SUB_E)])
        pltpu.sync_copy(rows_d, gd_hbm.at[pl.ds(base_e, SUB_E)])
        return ()

    lax.fori_loop(0, N_SUB, sub_step, (), unroll=False)


def _edge_body(gs_ref, gd_ref, he_ref, rv_ref, d8_ref,
               w1cat_ref, wetcat_ref, r_ref, snt_ref, s_ref,
               cnt_ref, cet_ref, bt2_ref,
               wsp1_ref, bsp1_ref, wsp2_ref, bsp2_ref, out_ref):
    f32 = jnp.float32
    dot = functools.partial(jnp.dot, preferred_element_type=f32)
    gs = gs_ref[...]
    s = gs[:, :DE]
    ssr = gs[:, DE:]
    d = gd_ref[...]
    R = r_ref[...]
    fnt = dot(dot(s, R) * dot(d, w1cat_ref[...]), snt_ref[...]) * (1.0 / 64.0) + cnt_ref[...]
    et2 = dot(he_ref[...], wetcat_ref[...]) * 0.25 + cet_ref[...]
    f2 = dot(dot(fnt, R) * et2, s_ref[...]) * (1.0 / 16.0) + bt2_ref[...]
    mu = jnp.mean(f2, axis=1, keepdims=True)
    c = f2 - mu
    var = jnp.mean(c * c, axis=1, keepdims=True)
    normed = c * lax.rsqrt(var + EPS)
    normed = normed * ssr[:, :DE] + ssr[:, DE:]
    h = dot(normed, wsp1_ref[...]) + bsp1_ref[...]
    h = h * (1.0 / (1.0 + jnp.exp(-h)))
    sw8 = dot(h, wsp2_ref[...]) + bsp2_ref[...]
    d8 = d8_ref[...]
    out_ref[...] = (sw8 / (d8 * (d8 + 1.0))) * rv_ref[...]


def _scatter_body(force_hbm, si_hbm, zeros_hbm, out_hbm,
                  acc, force_v, idx_v, sem):
    cid = lax.axis_index("c")
    tid = lax.axis_index("s")

    @pl.when(cid == 0)
    def _():
        pltpu.sync_copy(zeros_hbm.at[pl.ds(tid * 640, 640)],
                        acc.at[pl.ds(tid * 640, 640)])
        plsc.subcore_barrier()

        def sub_step(sub, _):
            base_row = tid * SC_ROWS_PER_T + sub * SC_SUB_ROWS
            pltpu.sync_copy(si_hbm.at[pl.ds(base_row, SC_SUB_ROWS)], idx_v)
            pltpu.sync_copy(force_hbm.at[pl.ds(base_row * 128, SC_SUB_ROWS * 128)],
                            force_v)
            for j in range(SC_SUB_ROWS):
                pltpu.sync_copy(force_v.at[pl.ds(j * 128, 128)],
                                acc.at[idx_v.at[j]], add=True)
            return ()

        lax.fori_loop(0, SC_N_SUB, sub_step, (), unroll=False)
        plsc.subcore_barrier()
        pltpu.sync_copy(acc.at[pl.ds(tid * 640, 640)],
                        out_hbm.at[pl.ds(tid * 640, 640)])


@jax.jit
def _run(h_node, h_edge, edge_index, relative_vec, distance, t, batch,
         W_src, b_src, W_dst, b_dst, w_tp1, b_tp1, W_nt, b_nt, W_et, b_et,
         w_tp2, b_tp2, W_t1, b_t1, W_t2, b_t2, W_sp1, b_sp1, W_sp2, b_sp2):
    f32 = jnp.float32

    ss_tab = pl.pallas_call(
        _tmlp_body,
        out_shape=jax.ShapeDtypeStruct((G, 2 * DE), f32),
    )(t.reshape(G, 1), W_t1, b_t1.reshape(1, T_DIM), W_t2, b_t2.reshape(1, 2 * DE))

    node_cat, node_dst = pl.pallas_call(
        _node_tf_body,
        grid=(10,),
        in_specs=[
            pl.BlockSpec((1000, D_NODE), lambda i: (i, 0)),
            pl.BlockSpec((1000, 1), lambda i: (i, 0)),
            pl.BlockSpec((D_NODE, DE), lambda i: (0, 0)),
            pl.BlockSpec((D_NODE, DE), lambda i: (0, 0)),
            pl.BlockSpec((1, DE), lambda i: (0, 0)),
            pl.BlockSpec((1, DE), lambda i: (0, 0)),
            pl.BlockSpec((G, 2 * DE), lambda i: (0, 0)),
        ],
        out_specs=[
            pl.BlockSpec((1000, DC), lambda i: (i, 0)),
            pl.BlockSpec((1000, DE), lambda i: (i, 0)),
        ],
        out_shape=[
            jax.ShapeDtypeStruct((N_NODES, DC), f32),
            jax.ShapeDtypeStruct((N_NODES, DE), f32),
        ],
    )(h_node, batch.reshape(N_NODES, 1), W_src, W_dst,
      b_src.reshape(1, DE), b_dst.reshape(1, DE), ss_tab)

    pad_e = E_PAD - N_EDGES
    src = jnp.concatenate([edge_index[0], jnp.zeros((pad_e,), jnp.int32)])
    dst = jnp.concatenate([edge_index[1], jnp.zeros((pad_e,), jnp.int32)])
    src3 = src.reshape(E_ROWS, 128)
    dst3 = dst.reshape(E_ROWS, 128)
    he_pad = jnp.concatenate([h_edge, jnp.zeros((pad_e, DE), f32)])
    rv_pad = jnp.concatenate([
        jnp.concatenate([relative_vec, jnp.zeros((N_EDGES, FW - 3), f32)], axis=1),
        jnp.zeros((pad_e, FW), f32)])
    d8_pad = jnp.concatenate([
        jnp.broadcast_to(distance, (N_EDGES, FW)),
        jnp.ones((pad_e, FW), f32)])

    mesh = plsc.VectorSubcoreMesh(core_axis_name="c", subcore_axis_name="s")
    gs, gd = pl.kernel(
        _gather_body,
        out_type=[
            jax.ShapeDtypeStruct((E_PAD, DC), f32),
            jax.ShapeDtypeStruct((E_PAD, DE), f32),
        ],
        mesh=mesh,
        scratch_types=[
            pltpu.VMEM((SUB_ROWS, 128), jnp.int32),
            pltpu.VMEM((SUB_ROWS, 128), jnp.int32),
            pltpu.VMEM((SUB_E, DC), f32),
            pltpu.VMEM((SUB_E, DE), f32),
            pltpu.SemaphoreType.DMA,
        ],
        compiler_params=pltpu.CompilerParams(use_tc_tiling_on_sc=False),
    )(node_cat, node_dst, src3, dst3)

    w1cat = w_tp1.transpose(1, 0, 2).reshape(DE, DE * DE)
    w2cat = w_tp2.transpose(1, 0, 2).reshape(DE, DE * DE)
    R = jnp.kron(jnp.eye(DE, dtype=f32), jnp.ones((1, DE), f32))
    S = jnp.kron(jnp.ones((DE, 1), f32), jnp.eye(DE, dtype=f32))
    S_nt = S @ W_nt
    c_nt = (b_tp1 @ W_nt * 0.25 + b_nt).reshape(1, DE)
    wetcat = W_et @ w2cat
    c_et = (b_et @ w2cat).reshape(1, DE * DE)
    wsp2_8 = jnp.tile(W_sp2, (1, FW))
    bsp2_8 = jnp.tile(b_sp2.reshape(1, 1), (1, FW))
    n_blk = E_PAD // BC
    force = pl.pallas_call(
        _edge_body,
        grid=(n_blk,),
        in_specs=[
            pl.BlockSpec((BC, DC), lambda i: (i, 0)),
            pl.BlockSpec((BC, DE), lambda i: (i, 0)),
            pl.BlockSpec((BC, DE), lambda i: (i, 0)),
            pl.BlockSpec((BC, FW), lambda i: (i, 0)),
            pl.BlockSpec((BC, FW), lambda i: (i, 0)),
            pl.BlockSpec((DE, DE * DE), lambda i: (0, 0)),
            pl.BlockSpec((DE, DE * DE), lambda i: (0, 0)),
            pl.BlockSpec((DE, DE * DE), lambda i: (0, 0)),
            pl.BlockSpec((DE * DE, DE), lambda i: (0, 0)),
            pl.BlockSpec((DE * DE, DE), lambda i: (0, 0)),
            pl.BlockSpec((1, DE), lambda i: (0, 0)),
            pl.BlockSpec((1, DE * DE), lambda i: (0, 0)),
            pl.BlockSpec((1, DE), lambda i: (0, 0)),
            pl.BlockSpec((DE, 32), lambda i: (0, 0)),
            pl.BlockSpec((1, 32), lambda i: (0, 0)),
            pl.BlockSpec((32, FW), lambda i: (0, 0)),
            pl.BlockSpec((1, FW), lambda i: (0, 0)),
        ],
        out_specs=pl.BlockSpec((BC, FW), lambda i: (i, 0)),
        out_shape=jax.ShapeDtypeStruct((E_PAD, FW), f32),
    )(gs, gd, he_pad, rv_pad, d8_pad,
      w1cat, wetcat, R, S_nt, S,
      c_nt, c_et, b_tp2.reshape(1, DE),
      W_sp1, b_sp1.reshape(1, 32), wsp2_8, bsp2_8)

    dp = pl.kernel(
        _scatter_body,
        out_type=jax.ShapeDtypeStruct((N_ACC, FW), f32),
        mesh=mesh,
        scratch_types=[
            pltpu.VMEM_SHARED((N_ACC, FW), f32),
            pltpu.VMEM((SC_SUB_ROWS * 128, FW), f32),
            pltpu.VMEM((SC_SUB_ROWS, 128), jnp.int32),
            pltpu.SemaphoreType.DMA,
        ],
        compiler_params=pltpu.CompilerParams(use_tc_tiling_on_sc=False),
    )(force, src3, jnp.zeros((N_ACC, FW), f32))

    return dp[:N_NODES, :3]


def kernel(h_node, h_edge, pos, edge_index, relative_vec, distance, t, batch,
           W_src, b_src, W_dst, b_dst, w_tp1, b_tp1, W_nt, b_nt, W_et, b_et,
           w_tp2, b_tp2, W_t1, b_t1, W_t2, b_t2, W_sp1, b_sp1, W_sp2, b_sp2):
    return _run(h_node, h_edge, edge_index, relative_vec, distance, t, batch,
                W_src, b_src, W_dst, b_dst, w_tp1, b_tp1, W_nt, b_nt, W_et,
                b_et, w_tp2, b_tp2, W_t1, b_t1, W_t2, b_t2, W_sp1, b_sp1,
                W_sp2, b_sp2)


# speedup vs baseline: 4.6238x; 1.0402x over previous
import functools

import jax
import jax.numpy as jnp
from jax import lax
from jax.experimental import pallas as pl
from jax.experimental.pallas import tpu as pltpu
from jax.experimental.pallas import tpu_sc as plsc

N_NODES = 10000
N_EDGES = 320000
D_NODE = 128
DE = 16
DC = 48
G = 32
T_DIM = 128
EPS = 1e-5

NC = 2
NS = 16
NW = NC * NS

E_ROWS = 2560
E_PAD = E_ROWS * 128
ROWS_PER_W = E_ROWS // NW
SUB_ROWS = 8
N_SUB = ROWS_PER_W // SUB_ROWS
SUB_E = SUB_ROWS * 128

N_ACC = 10240
SC_ROWS_PER_T = E_ROWS // NS
SC_SUB_ROWS = 16
SC_N_SUB = SC_ROWS_PER_T // SC_SUB_ROWS

BC = 4096
FW = 8


def _node_tf_body(h_ref, batch_ref, ws_ref, wd_ref, bs_ref, bd_ref, ss_ref,
                  ncat_ref, nd_ref):
    h = h_ref[...]
    inv = 1.0 / jnp.sqrt(float(D_NODE))
    src = jnp.dot(h, ws_ref[...], preferred_element_type=jnp.float32) * inv + bs_ref[...]
    nd_ref[...] = jnp.dot(h, wd_ref[...], preferred_element_type=jnp.float32) * inv + bd_ref[...]
    onehot = (batch_ref[...] == lax.broadcasted_iota(jnp.int32, (h.shape[0], G), 1)).astype(jnp.float32)
    ss = jnp.dot(onehot, ss_ref[...], preferred_element_type=jnp.float32)
    ncat_ref[...] = jnp.concatenate([src, ss], axis=1)


def _tmlp_body(t_ref, w1_ref, b1_ref, w2_ref, b2_ref, ss_ref):
    half = T_DIM // 2
    k = lax.broadcasted_iota(jnp.int32, (G, half), 1).astype(jnp.float32)
    freqs = jnp.exp(k * (-jnp.log(10000.0) / half))
    args = t_ref[...] * freqs
    temb = jnp.concatenate([jnp.cos(args), jnp.sin(args)], axis=1)
    x = jnp.dot(temb, w1_ref[...], preferred_element_type=jnp.float32) + b1_ref[...]
    x = x * (1.0 / (1.0 + jnp.exp(-x)))
    ss = jnp.dot(x, w2_ref[...], preferred_element_type=jnp.float32) + b2_ref[...]
    one0 = jnp.concatenate([jnp.ones((1, DE), jnp.float32), jnp.zeros((1, DE), jnp.float32)], axis=1)
    ss_ref[...] = ss + one0


def _gather_body(nc_hbm, nd_hbm, si_hbm, di_hbm,
                 gs_hbm, gd_hbm,
                 idx_s, idx_d, rows_s, rows_d, sem):
    wid = lax.axis_index("s") * NC + lax.axis_index("c")

    def sub_step(sub, _):
        base_row = wid * ROWS_PER_W + sub * SUB_ROWS
        pltpu.sync_copy(si_hbm.at[pl.ds(base_row, SUB_ROWS)], idx_s)
        pltpu.sync_copy(di_hbm.at[pl.ds(base_row, SUB_ROWS)], idx_d)
        descs = []
        for j in range(SUB_ROWS):
            descs.append(pltpu.async_copy(
                nc_hbm.at[idx_s.at[j]], rows_s.at[pl.ds(j * 128, 128)], sem))
            descs.append(pltpu.async_copy(
                nd_hbm.at[idx_d.at[j]], rows_d.at[pl.ds(j * 128, 128)], sem))
        for dsc in descs:
            dsc.wait()
        base_e = base_row * 128
        pltpu.sync_copy(rows_s, gs_hbm.at[pl.ds(base_e, SUB_E)])
        pltpu.sync_copy(rows_d, gd_hbm.at[pl.ds(base_e, SUB_E)])
        return ()

    lax.fori_loop(0, N_SUB, sub_step, (), unroll=False)


def _edge_body(gs_ref, gd_ref, he_ref, rv_ref, d8_ref,
               w1cat_ref, wetcat_ref, r_ref, snt_ref, s_ref,
               cnt_ref, cet_ref, bt2_ref,
               wsp1_ref, bsp1_ref, wsp2_ref, bsp2_ref, out_ref):
    f32 = jnp.float32
    dot = functools.partial(jnp.dot, preferred_element_type=f32)
    gs = gs_ref[...]
    s = gs[:, :DE]
    ssr = gs[:, DE:]
    d = gd_ref[...]
    R = r_ref[...]
    fnt = dot(dot(s, R) * dot(d, w1cat_ref[...]), snt_ref[...]) * (1.0 / 64.0) + cnt_ref[...]
    et2 = dot(he_ref[...], wetcat_ref[...]) * 0.25 + cet_ref[...]
    f2 = dot(dot(fnt, R) * et2, s_ref[...]) * (1.0 / 16.0) + bt2_ref[...]
    mu = jnp.mean(f2, axis=1, keepdims=True)
    c = f2 - mu
    var = jnp.mean(c * c, axis=1, keepdims=True)
    normed = c * lax.rsqrt(var + EPS)
    normed = normed * ssr[:, :DE] + ssr[:, DE:]
    h = dot(normed, wsp1_ref[...]) + bsp1_ref[...]
    h = h * (1.0 / (1.0 + jnp.exp(-h)))
    sw8 = dot(h, wsp2_ref[...]) + bsp2_ref[...]
    d8 = d8_ref[...]
    out_ref[...] = (sw8 / (d8 * (d8 + 1.0))) * rv_ref[...]


def _scatter_body(force_hbm, si_hbm, zeros_hbm, out_hbm,
                  acc, force_v, idx_v, sem):
    cid = lax.axis_index("c")
    tid = lax.axis_index("s")

    @pl.when(cid == 0)
    def _():
        pltpu.sync_copy(zeros_hbm.at[pl.ds(tid * 640, 640)],
                        acc.at[pl.ds(tid * 640, 640)])
        plsc.subcore_barrier()

        def sub_step(sub, _):
            base_row = tid * SC_ROWS_PER_T + sub * SC_SUB_ROWS
            pltpu.sync_copy(si_hbm.at[pl.ds(base_row, SC_SUB_ROWS)], idx_v)
            pltpu.sync_copy(force_hbm.at[pl.ds(base_row * 128, SC_SUB_ROWS * 128)],
                            force_v)
            for j in range(SC_SUB_ROWS):
                pltpu.sync_copy(force_v.at[pl.ds(j * 128, 128)],
                                acc.at[idx_v.at[j]], add=True)
            return ()

        lax.fori_loop(0, SC_N_SUB, sub_step, (), unroll=False)
        plsc.subcore_barrier()
        pltpu.sync_copy(acc.at[pl.ds(tid * 640, 640)],
                        out_hbm.at[pl.ds(tid * 640, 640)])


@jax.jit
def _run(h_node, h_edge, edge_index, relative_vec, distance, t, batch,
         W_src, b_src, W_dst, b_dst, w_tp1, b_tp1, W_nt, b_nt, W_et, b_et,
         w_tp2, b_tp2, W_t1, b_t1, W_t2, b_t2, W_sp1, b_sp1, W_sp2, b_sp2):
    f32 = jnp.float32

    ss_tab = pl.pallas_call(
        _tmlp_body,
        out_shape=jax.ShapeDtypeStruct((G, 2 * DE), f32),
    )(t.reshape(G, 1), W_t1, b_t1.reshape(1, T_DIM), W_t2, b_t2.reshape(1, 2 * DE))

    node_cat, node_dst = pl.pallas_call(
        _node_tf_body,
        grid=(10,),
        in_specs=[
            pl.BlockSpec((1000, D_NODE), lambda i: (i, 0)),
            pl.BlockSpec((1000, 1), lambda i: (i, 0)),
            pl.BlockSpec((D_NODE, DE), lambda i: (0, 0)),
            pl.BlockSpec((D_NODE, DE), lambda i: (0, 0)),
            pl.BlockSpec((1, DE), lambda i: (0, 0)),
            pl.BlockSpec((1, DE), lambda i: (0, 0)),
            pl.BlockSpec((G, 2 * DE), lambda i: (0, 0)),
        ],
        out_specs=[
            pl.BlockSpec((1000, DC), lambda i: (i, 0)),
            pl.BlockSpec((1000, DE), lambda i: (i, 0)),
        ],
        out_shape=[
            jax.ShapeDtypeStruct((N_NODES, DC), f32),
            jax.ShapeDtypeStruct((N_NODES, DE), f32),
        ],
    )(h_node, batch.reshape(N_NODES, 1), W_src, W_dst,
      b_src.reshape(1, DE), b_dst.reshape(1, DE), ss_tab)

    pad_e = E_PAD - N_EDGES
    src = jnp.concatenate([edge_index[0], jnp.zeros((pad_e,), jnp.int32)])
    dst = jnp.concatenate([edge_index[1], jnp.zeros((pad_e,), jnp.int32)])
    src3 = src.reshape(E_ROWS, 128)
    dst3 = dst.reshape(E_ROWS, 128)
    he_pad = jnp.concatenate([h_edge, jnp.zeros((pad_e, DE), f32)])
    rv_pad = jnp.concatenate([
        jnp.concatenate([relative_vec, jnp.zeros((N_EDGES, FW - 3), f32)], axis=1),
        jnp.zeros((pad_e, FW), f32)])
    d8_pad = jnp.concatenate([
        jnp.broadcast_to(distance, (N_EDGES, FW)),
        jnp.ones((pad_e, FW), f32)])

    mesh = plsc.VectorSubcoreMesh(core_axis_name="c", subcore_axis_name="s")
    gs, gd = pl.kernel(
        _gather_body,
        out_type=[
            jax.ShapeDtypeStruct((E_PAD, DC), f32),
            jax.ShapeDtypeStruct((E_PAD, DE), f32),
        ],
        mesh=mesh,
        scratch_types=[
            pltpu.VMEM((SUB_ROWS, 128), jnp.int32),
            pltpu.VMEM((SUB_ROWS, 128), jnp.int32),
            pltpu.VMEM((SUB_E, DC), f32),
            pltpu.VMEM((SUB_E, DE), f32),
            pltpu.SemaphoreType.DMA,
        ],
        compiler_params=pltpu.CompilerParams(use_tc_tiling_on_sc=False),
    )(node_cat, node_dst, src3, dst3)

    w1cat = w_tp1.transpose(1, 0, 2).reshape(DE, DE * DE)
    w2cat = w_tp2.transpose(1, 0, 2).reshape(DE, DE * DE)
    R = jnp.kron(jnp.eye(DE, dtype=f32), jnp.ones((1, DE), f32))
    S = jnp.kron(jnp.ones((DE, 1), f32), jnp.eye(DE, dtype=f32))
    S_nt = S @ W_nt
    c_nt = (b_tp1 @ W_nt * 0.25 + b_nt).reshape(1, DE)
    wetcat = W_et @ w2cat
    c_et = (b_et @ w2cat).reshape(1, DE * DE)
    wsp2_8 = jnp.tile(W_sp2, (1, FW))
    bsp2_8 = jnp.tile(b_sp2.reshape(1, 1), (1, FW))
    n_blk = E_PAD // BC
    force = pl.pallas_call(
        _edge_body,
        grid=(n_blk,),
        in_specs=[
            pl.BlockSpec((BC, DC), lambda i: (i, 0)),
            pl.BlockSpec((BC, DE), lambda i: (i, 0)),
            pl.BlockSpec((BC, DE), lambda i: (i, 0)),
            pl.BlockSpec((BC, FW), lambda i: (i, 0)),
            pl.BlockSpec((BC, FW), lambda i: (i, 0)),
            pl.BlockSpec((DE, DE * DE), lambda i: (0, 0)),
            pl.BlockSpec((DE, DE * DE), lambda i: (0, 0)),
            pl.BlockSpec((DE, DE * DE), lambda i: (0, 0)),
            pl.BlockSpec((DE * DE, DE), lambda i: (0, 0)),
            pl.BlockSpec((DE * DE, DE), lambda i: (0, 0)),
            pl.BlockSpec((1, DE), lambda i: (0, 0)),
            pl.BlockSpec((1, DE * DE), lambda i: (0, 0)),
            pl.BlockSpec((1, DE), lambda i: (0, 0)),
            pl.BlockSpec((DE, 32), lambda i: (0, 0)),
            pl.BlockSpec((1, 32), lambda i: (0, 0)),
            pl.BlockSpec((32, FW), lambda i: (0, 0)),
            pl.BlockSpec((1, FW), lambda i: (0, 0)),
        ],
        out_specs=pl.BlockSpec((BC, FW), lambda i: (i, 0)),
        out_shape=jax.ShapeDtypeStruct((E_PAD, FW), f32),
    )(gs, gd, he_pad, rv_pad, d8_pad,
      w1cat, wetcat, R, S_nt, S,
      c_nt, c_et, b_tp2.reshape(1, DE),
      W_sp1, b_sp1.reshape(1, 32), wsp2_8, bsp2_8)

    dp = pl.kernel(
        _scatter_body,
        out_type=jax.ShapeDtypeStruct((N_ACC, FW), f32),
        mesh=mesh,
        scratch_types=[
            pltpu.VMEM_SHARED((N_ACC, FW), f32),
            pltpu.VMEM((SC_SUB_ROWS * 128, FW), f32),
            pltpu.VMEM((SC_SUB_ROWS, 128), jnp.int32),
            pltpu.SemaphoreType.DMA,
        ],
        compiler_params=pltpu.CompilerParams(use_tc_tiling_on_sc=False),
    )(force, src3, jnp.zeros((N_ACC, FW), f32))

    return dp[:N_NODES, :3]


def kernel(h_node, h_edge, pos, edge_index, relative_vec, distance, t, batch,
           W_src, b_src, W_dst, b_dst, w_tp1, b_tp1, W_nt, b_nt, W_et, b_et,
           w_tp2, b_tp2, W_t1, b_t1, W_t2, b_t2, W_sp1, b_sp1, W_sp2, b_sp2):
    return _run(h_node, h_edge, edge_index, relative_vec, distance, t, batch,
                W_src, b_src, W_dst, b_dst, w_tp1, b_tp1, W_nt, b_nt, W_et,
                b_et, w_tp2, b_tp2, W_t1, b_t1, W_t2, b_t2, W_sp1, b_sp1,
                W_sp2, b_sp2)
